# baseline (speedup 1.0000x reference)
import jax
import jax.numpy as jnp
from jax import lax
from jax.experimental import pallas as pl
from jax.experimental.pallas import tpu as pltpu
from jax.experimental.pallas import tpu_sc as plsc

_NB, _NR, _NCOL = 512, 512, 256
_RS = 16
_CS = 8
_NTR = _NR // _RS
_NW = 32
_BPW = _NB // _NW
_LANES = 16


def _sc_body(x_hbm, o_hbm, rows_v, sem_b, sem_g, sem_s):
    wid = lax.axis_index("s") * 2 + lax.axis_index("c")
    base = wid * _BPW
    bulk = []
    for g in range(_BPW):
        cp = pltpu.make_async_copy(x_hbm.at[base + g], o_hbm.at[base + g], sem_b)
        cp.start()
        bulk.append(cp)
    lanes = lax.iota(jnp.int32, _LANES)
    keep = jnp.where(lanes % _CS == 0,
                     jnp.int32(-0x40000001), jnp.int32(-1))
    for g in range(_BPW):
        b = base + g
        gcp = pltpu.make_async_copy(x_hbm.at[b, :, 0], rows_v, sem_g)
        gcp.start()
        gcp.wait()

        def _mask_row(r, carry):
            for j in range(_NCOL // _LANES):
                v = rows_v[r, pl.ds(j * _LANES, _LANES)]
                u = lax.bitcast_convert_type(v, jnp.int32) & keep
                rows_v[r, pl.ds(j * _LANES, _LANES)] = lax.bitcast_convert_type(u, jnp.float32)
            return carry

        lax.fori_loop(0, _NTR, _mask_row, 0)
        bulk[g].wait()
        scp = pltpu.make_async_copy(rows_v, o_hbm.at[b, :, 0], sem_s)
        scp.start()
        scp.wait()


def kernel(x):
    x4 = x.reshape(_NB, _NTR, _RS, _NCOL)
    run = pl.kernel(
        _sc_body,
        out_type=jax.ShapeDtypeStruct((_NB, _NTR, _RS, _NCOL), jnp.float32),
        mesh=plsc.VectorSubcoreMesh(core_axis_name="c", subcore_axis_name="s"),
        scratch_types=[
            pltpu.VMEM((_NTR, _NCOL), jnp.float32),
            pltpu.SemaphoreType.DMA,
            pltpu.SemaphoreType.DMA,
            pltpu.SemaphoreType.DMA,
        ],
    )
    return run(x4).reshape(x.shape)


# speedup vs baseline: 39.7167x; 39.7167x over previous
import jax
import jax.numpy as jnp
from jax import lax
from jax.experimental import pallas as pl
from jax.experimental.pallas import tpu as pltpu
from jax.experimental.pallas import tpu_sc as plsc

_NB, _NR, _NCOL = 512, 512, 256
_RS = 16
_CS = 8
_LANES = 16
_FLATROWS = _NB * _NR
_NW = 32
_RPW = _FLATROWS // _NW
_CH = 32
_NCH = _RPW // _CH
_NBUF = 8
_PF = 4

_MUTE = jnp.int32(-0x40000001)
_KEEPALL = jnp.int32(-1)


def _sc_body(x_hbm, o_hbm, bufs, *sems):
    sin = sems[:_NBUF]
    sout = sems[_NBUF:]
    wid = lax.axis_index("s") * 2 + lax.axis_index("c")
    wbase = wid * _RPW

    lanes = lax.iota(jnp.int32, _LANES)
    keep = jnp.where(lanes % _CS == 0, _MUTE, _KEEPALL)

    def start_in(row0, k):
        pltpu.make_async_copy(
            x_hbm.at[pl.ds(row0, _CH)], bufs.at[k], sin[k]).start()

    def wait_in(k):
        pltpu.make_async_copy(
            x_hbm.at[pl.ds(0, _CH)], bufs.at[k], sin[k]).wait()

    def start_out(row0, k):
        pltpu.make_async_copy(
            bufs.at[k], o_hbm.at[pl.ds(row0, _CH)], sout[k]).start()

    def wait_out(k):
        pltpu.make_async_copy(
            bufs.at[k], o_hbm.at[pl.ds(0, _CH)], sout[k]).wait()

    def mask(k):
        for r in range(0, _CH, _RS):
            for j in range(_NCOL // _LANES):
                v = bufs[k, r, pl.ds(j * _LANES, _LANES)]
                u = lax.bitcast_convert_type(v, jnp.int32) & keep
                bufs[k, r, pl.ds(j * _LANES, _LANES)] = (
                    lax.bitcast_convert_type(u, jnp.float32))

    for i in range(_PF):
        start_in(wbase + i * _CH, i)
    for i in range(_PF):
        wait_in(i)
        mask(i)
        start_out(wbase + i * _CH, i)
        start_in(wbase + (i + _PF) * _CH, (i + _PF) % _NBUF)

    ngroups = (_NCH - 2 * _PF) // _NBUF

    def group(g, carry):
        for k in range(_NBUF):
            i_off = _PF + k
            row0 = wbase + (i_off * _CH) + g * (_NBUF * _CH)
            kb = i_off % _NBUF
            wait_in(kb)
            mask(kb)
            start_out(row0, kb)
            kp = (i_off + _PF) % _NBUF
            wait_out(kp)
            start_in(row0 + _PF * _CH, kp)
        return carry

    lax.fori_loop(0, ngroups, group, 0)

    for e in range(_PF):
        i = _NCH - _PF + e
        kb = i % _NBUF
        row0 = wbase + i * _CH
        wait_in(kb)
        mask(kb)
        start_out(row0, kb)

    for k in range(_NBUF):
        wait_out(k)


def kernel(x):
    x2 = x.reshape(_FLATROWS, _NCOL)
    run = pl.kernel(
        _sc_body,
        out_type=jax.ShapeDtypeStruct((_FLATROWS, _NCOL), jnp.float32),
        mesh=plsc.VectorSubcoreMesh(core_axis_name="c", subcore_axis_name="s"),
        scratch_types=(
            [pltpu.VMEM((_NBUF, _CH, _NCOL), jnp.float32)]
            + [pltpu.SemaphoreType.DMA] * (2 * _NBUF)
        ),
    )
    return run(x2).reshape(x.shape)


# speedup vs baseline: 39.7292x; 1.0003x over previous
import jax
import jax.numpy as jnp
from jax import lax
from jax.experimental import pallas as pl
from jax.experimental.pallas import tpu as pltpu
from jax.experimental.pallas import tpu_sc as plsc

_NB, _NR, _NCOL = 512, 512, 256
_RS = 16
_CS = 8
_LANES = 16
_FLATROWS = _NB * _NR
_NW = 32
_RPW = _FLATROWS // _NW

_CH = 64
_NBUF = 4
_PF = 2
_NCH = _RPW // _CH

_MUTE = jnp.int32(-0x40000001)
_KEEPALL = jnp.int32(-1)


def _sc_body(x_hbm, o_hbm, bufs, *sems):
    sin = sems[:_NBUF]
    sout = sems[_NBUF:]
    wid = lax.axis_index("s") * 2 + lax.axis_index("c")
    wbase = wid * _RPW

    lanes = lax.iota(jnp.int32, _LANES)
    keep = jnp.where(lanes % _CS == 0, _MUTE, _KEEPALL)

    def start_in(row0, k):
        pltpu.make_async_copy(
            x_hbm.at[pl.ds(row0, _CH)], bufs.at[k], sin[k]).start()

    def wait_in(k):
        pltpu.make_async_copy(
            x_hbm.at[pl.ds(0, _CH)], bufs.at[k], sin[k]).wait()

    def start_out(row0, k):
        pltpu.make_async_copy(
            bufs.at[k], o_hbm.at[pl.ds(row0, _CH)], sout[k]).start()

    def wait_out(k):
        pltpu.make_async_copy(
            bufs.at[k], o_hbm.at[pl.ds(0, _CH)], sout[k]).wait()

    def mask(k):
        for r in range(0, _CH, _RS):
            for j in range(_NCOL // _LANES):
                v = bufs[k, r, pl.ds(j * _LANES, _LANES)]
                u = lax.bitcast_convert_type(v, jnp.int32) & keep
                bufs[k, r, pl.ds(j * _LANES, _LANES)] = (
                    lax.bitcast_convert_type(u, jnp.float32))

    def step(row0, kb, kp, prefetch, waitout):
        wait_in(kb)
        mask(kb)
        start_out(row0, kb)
        if prefetch:
            if waitout:
                wait_out(kp)
            start_in(row0 + _PF * _CH, kp)

    for i in range(_PF):
        start_in(wbase + i * _CH, i)
    for i in range(_PF):
        step(wbase + i * _CH, i % _NBUF, (i + _PF) % _NBUF,
             prefetch=True, waitout=(i + _PF) >= _NBUF)

    ngroups = (_NCH - 2 * _PF) // _NBUF
    main_end = _PF + ngroups * _NBUF

    def group(g, carry):
        for k in range(_NBUF):
            i_off = _PF + k
            row0 = wbase + i_off * _CH + g * (_NBUF * _CH)
            step(row0, i_off % _NBUF, (i_off + _PF) % _NBUF,
                 prefetch=True, waitout=True)
        return carry

    lax.fori_loop(0, ngroups, group, 0)

    for i in range(main_end, _NCH):
        step(wbase + i * _CH, i % _NBUF, (i + _PF) % _NBUF,
             prefetch=(i + _PF) < _NCH, waitout=True)

    for k in range(_NBUF):
        wait_out(k)


def kernel(x):
    x2 = x.reshape(_FLATROWS, _NCOL)
    run = pl.kernel(
        _sc_body,
        out_type=jax.ShapeDtypeStruct((_FLATROWS, _NCOL), jnp.float32),
        mesh=plsc.VectorSubcoreMesh(core_axis_name="c", subcore_axis_name="s"),
        scratch_types=(
            [pltpu.VMEM((_NBUF, _CH, _NCOL), jnp.float32)]
            + [pltpu.SemaphoreType.DMA] * (2 * _NBUF)
        ),
    )
    return run(x2).reshape(x.shape)


# speedup vs baseline: 40.7695x; 1.0262x over previous
import jax
import jax.numpy as jnp
from jax import lax
from jax.experimental import pallas as pl
from jax.experimental.pallas import tpu as pltpu
from jax.experimental.pallas import tpu_sc as plsc

_NB, _NR, _NCOL = 512, 512, 256
_RS = 16
_CS = 8
_LANES = 16
_FLATROWS = _NB * _NR
_NTOUCH = _FLATROWS // _RS
_NW = 32
_TPW = _NTOUCH // _NW

_CH = 64
_NBUF = 4
_PF = 2
_NCH = _TPW // _CH

_MUTE = jnp.int32(-0x40000001)

_TCB = 8


def _tc_copy_body(x_ref, o_ref):
    o_ref[...] = x_ref[...]


def _sc_body(x_hbm, o_hbm, bufs, *sems):
    sin = sems[:_NBUF]
    sout = sems[_NBUF:]
    wid = lax.axis_index("s") * 2 + lax.axis_index("c")
    wbase = wid * _TPW

    colbase = _CS * lax.iota(jnp.int32, _LANES)

    def start_in(t0, k):
        pltpu.make_async_copy(
            x_hbm.at[pl.ds(t0, _CH), 0], bufs.at[k], sin[k]).start()

    def wait_in(k):
        pltpu.make_async_copy(
            x_hbm.at[pl.ds(0, _CH), 0], bufs.at[k], sin[k]).wait()

    def start_out(t0, k):
        pltpu.make_async_copy(
            bufs.at[k], o_hbm.at[pl.ds(t0, _CH), 0], sout[k]).start()

    def wait_out(k):
        pltpu.make_async_copy(
            bufs.at[k], o_hbm.at[pl.ds(0, _CH), 0], sout[k]).wait()

    def mask(kb):
        kvec = jnp.full((_LANES,), kb, jnp.int32)

        def row_body(r, carry):
            rvec = jnp.full((_LANES,), r, jnp.int32)
            for off in (0, 128):
                idx = [kvec, rvec, colbase + off]
                v = plsc.load_gather(bufs, idx)
                u = lax.bitcast_convert_type(v, jnp.int32) & _MUTE
                plsc.store_scatter(
                    bufs, idx, lax.bitcast_convert_type(u, jnp.float32))
            return carry

        lax.fori_loop(0, _CH, row_body, 0)

    def step(t0, kb, kp, prefetch, waitout):
        wait_in(kb)
        mask(kb)
        start_out(t0, kb)
        if prefetch:
            if waitout:
                wait_out(kp)
            start_in(t0 + _PF * _CH, kp)

    for i in range(_PF):
        start_in(wbase + i * _CH, i)
    for i in range(_PF):
        step(wbase + i * _CH, i % _NBUF, (i + _PF) % _NBUF,
             prefetch=True, waitout=(i + _PF) >= _NBUF)

    ngroups = (_NCH - 2 * _PF) // _NBUF
    main_end = _PF + ngroups * _NBUF

    def group(g, carry):
        for k in range(_NBUF):
            i_off = _PF + k
            t0 = wbase + i_off * _CH + g * (_NBUF * _CH)
            step(t0, i_off % _NBUF, (i_off + _PF) % _NBUF,
                 prefetch=True, waitout=True)
        return carry

    lax.fori_loop(0, ngroups, group, 0)

    for i in range(main_end, _NCH):
        step(wbase + i * _CH, i % _NBUF, (i + _PF) % _NBUF,
             prefetch=(i + _PF) < _NCH, waitout=True)

    for k in range(_NBUF):
        wait_out(k)


def kernel(x):
    out0 = pl.pallas_call(
        _tc_copy_body,
        grid=(_NB // _TCB,),
        in_specs=[pl.BlockSpec((_TCB, _NR, _NCOL), lambda i: (i, 0, 0))],
        out_specs=pl.BlockSpec((_TCB, _NR, _NCOL), lambda i: (i, 0, 0)),
        out_shape=jax.ShapeDtypeStruct(x.shape, x.dtype),
    )(x)

    x5 = x.reshape(_NTOUCH, _RS, _NCOL)
    out_r = jax.new_ref(out0.reshape(_NTOUCH, _RS, _NCOL))

    run = pl.kernel(
        _sc_body,
        out_type=(),
        mesh=plsc.VectorSubcoreMesh(core_axis_name="c", subcore_axis_name="s"),
        compiler_params=pltpu.CompilerParams(needs_layout_passes=False),
        scratch_types=(
            [pltpu.VMEM((_NBUF, _CH, _NCOL), jnp.float32)]
            + [pltpu.SemaphoreType.DMA] * (2 * _NBUF)
        ),
    )
    run(x5, out_r)
    return out_r[...].reshape(x.shape)


# speedup vs baseline: 41.0849x; 1.0077x over previous
import jax
import jax.numpy as jnp
from jax import lax
from jax.experimental import pallas as pl
from jax.experimental.pallas import tpu as pltpu
from jax.experimental.pallas import tpu_sc as plsc

_NB, _NR, _NCOL = 512, 512, 256
_RS = 16
_CS = 8
_LANES = 16
_FLATROWS = _NB * _NR
_NTOUCH = _FLATROWS // _RS
_NW = 32
_TPW = _NTOUCH // _NW

_CH = 64
_NBUF = 4
_PF = 2
_NCH = _TPW // _CH

_MUTE = jnp.int32(-0x40000001)

_TCB = 16


def _tc_copy_body(x_ref, o_ref):
    o_ref[...] = x_ref[...]


def _sc_body(x_hbm, o_hbm, bufs, *sems):
    sin = sems[:_NBUF]
    sout = sems[_NBUF:]
    wid = lax.axis_index("s") * 2 + lax.axis_index("c")
    wbase = wid * _TPW

    colbase = _CS * lax.iota(jnp.int32, _LANES)

    def start_in(t0, k):
        pltpu.make_async_copy(
            x_hbm.at[pl.ds(t0, _CH), 0], bufs.at[k], sin[k]).start()

    def wait_in(k):
        pltpu.make_async_copy(
            x_hbm.at[pl.ds(0, _CH), 0], bufs.at[k], sin[k]).wait()

    def start_out(t0, k):
        pltpu.make_async_copy(
            bufs.at[k], o_hbm.at[pl.ds(t0, _CH), 0], sout[k]).start()

    def wait_out(k):
        pltpu.make_async_copy(
            bufs.at[k], o_hbm.at[pl.ds(0, _CH), 0], sout[k]).wait()

    def mask(kb):
        kvec = jnp.full((_LANES,), kb, jnp.int32)

        def row_body(r, carry):
            rvec = jnp.full((_LANES,), r, jnp.int32)
            for off in (0, 128):
                idx = [kvec, rvec, colbase + off]
                v = plsc.load_gather(bufs, idx)
                u = lax.bitcast_convert_type(v, jnp.int32) & _MUTE
                plsc.store_scatter(
                    bufs, idx, lax.bitcast_convert_type(u, jnp.float32))
            return carry

        lax.fori_loop(0, _CH, row_body, 0)

    def step(t0, kb, kp, prefetch, waitout):
        wait_in(kb)
        mask(kb)
        start_out(t0, kb)
        if prefetch:
            if waitout:
                wait_out(kp)
            start_in(t0 + _PF * _CH, kp)

    for i in range(_PF):
        start_in(wbase + i * _CH, i)
    for i in range(_PF):
        step(wbase + i * _CH, i % _NBUF, (i + _PF) % _NBUF,
             prefetch=True, waitout=(i + _PF) >= _NBUF)

    ngroups = (_NCH - 2 * _PF) // _NBUF
    main_end = _PF + ngroups * _NBUF

    def group(g, carry):
        for k in range(_NBUF):
            i_off = _PF + k
            t0 = wbase + i_off * _CH + g * (_NBUF * _CH)
            step(t0, i_off % _NBUF, (i_off + _PF) % _NBUF,
                 prefetch=True, waitout=True)
        return carry

    lax.fori_loop(0, ngroups, group, 0)

    for i in range(main_end, _NCH):
        step(wbase + i * _CH, i % _NBUF, (i + _PF) % _NBUF,
             prefetch=(i + _PF) < _NCH, waitout=True)

    for k in range(_NBUF):
        wait_out(k)


def kernel(x):
    out0 = pl.pallas_call(
        _tc_copy_body,
        grid=(_NB // _TCB,),
        in_specs=[pl.BlockSpec((_TCB, _NR, _NCOL), lambda i: (i, 0, 0))],
        out_specs=pl.BlockSpec((_TCB, _NR, _NCOL), lambda i: (i, 0, 0)),
        out_shape=jax.ShapeDtypeStruct(x.shape, x.dtype),
    )(x)

    x5 = x.reshape(_NTOUCH, _RS, _NCOL)
    out_r = jax.new_ref(out0.reshape(_NTOUCH, _RS, _NCOL))

    run = pl.kernel(
        _sc_body,
        out_type=(),
        mesh=plsc.VectorSubcoreMesh(core_axis_name="c", subcore_axis_name="s"),
        compiler_params=pltpu.CompilerParams(needs_layout_passes=False),
        scratch_types=(
            [pltpu.VMEM((_NBUF, _CH, _NCOL), jnp.float32)]
            + [pltpu.SemaphoreType.DMA] * (2 * _NBUF)
        ),
    )
    run(x5, out_r)
    return out_r[...].reshape(x.shape)


# speedup vs baseline: 41.2453x; 1.0039x over previous
import jax
import jax.numpy as jnp
from jax import lax
from jax.experimental import pallas as pl
from jax.experimental.pallas import tpu as pltpu
from jax.experimental.pallas import tpu_sc as plsc

_NB, _NR, _NCOL = 512, 512, 256
_RS = 16
_CS = 8
_LANES = 16
_FLATROWS = _NB * _NR
_NTOUCH = _FLATROWS // _RS
_NW = 32
_TPW = _NTOUCH // _NW

_CH = 32
_NBUF = 8
_PF = 4
_NCH = _TPW // _CH

_MUTE = jnp.int32(-0x40000001)

_TCB = 16


def _tc_copy_body(x_ref, o_ref):
    o_ref[...] = x_ref[...]


def _sc_body(x_hbm, o_hbm, bufs, *sems):
    sin = sems[:_NBUF]
    sout = sems[_NBUF:]
    wid = lax.axis_index("s") * 2 + lax.axis_index("c")
    wbase = wid * _TPW

    colbase = _CS * lax.iota(jnp.int32, _LANES)

    def start_in(t0, k):
        pltpu.make_async_copy(
            x_hbm.at[pl.ds(t0, _CH), 0], bufs.at[k], sin[k]).start()

    def wait_in(k):
        pltpu.make_async_copy(
            x_hbm.at[pl.ds(0, _CH), 0], bufs.at[k], sin[k]).wait()

    def start_out(t0, k):
        pltpu.make_async_copy(
            bufs.at[k], o_hbm.at[pl.ds(t0, _CH), 0], sout[k]).start()

    def wait_out(k):
        pltpu.make_async_copy(
            bufs.at[k], o_hbm.at[pl.ds(0, _CH), 0], sout[k]).wait()

    def mask(kb):
        kvec = jnp.full((_LANES,), kb, jnp.int32)

        def row_body(r, carry):
            rvec = jnp.full((_LANES,), r, jnp.int32)
            for off in (0, 128):
                idx = [kvec, rvec, colbase + off]
                v = plsc.load_gather(bufs, idx)
                u = lax.bitcast_convert_type(v, jnp.int32) & _MUTE
                plsc.store_scatter(
                    bufs, idx, lax.bitcast_convert_type(u, jnp.float32))
            return carry

        lax.fori_loop(0, _CH, row_body, 0)

    def step(t0, kb, kp, prefetch, waitout):
        wait_in(kb)
        mask(kb)
        start_out(t0, kb)
        if prefetch:
            if waitout:
                wait_out(kp)
            start_in(t0 + _PF * _CH, kp)

    for i in range(_PF):
        start_in(wbase + i * _CH, i)
    for i in range(_PF):
        step(wbase + i * _CH, i % _NBUF, (i + _PF) % _NBUF,
             prefetch=True, waitout=(i + _PF) >= _NBUF)

    ngroups = (_NCH - 2 * _PF) // _NBUF
    main_end = _PF + ngroups * _NBUF

    def group(g, carry):
        for k in range(_NBUF):
            i_off = _PF + k
            t0 = wbase + i_off * _CH + g * (_NBUF * _CH)
            step(t0, i_off % _NBUF, (i_off + _PF) % _NBUF,
                 prefetch=True, waitout=True)
        return carry

    lax.fori_loop(0, ngroups, group, 0)

    for i in range(main_end, _NCH):
        step(wbase + i * _CH, i % _NBUF, (i + _PF) % _NBUF,
             prefetch=(i + _PF) < _NCH, waitout=True)

    for k in range(_NBUF):
        wait_out(k)


def kernel(x):
    out0 = pl.pallas_call(
        _tc_copy_body,
        grid=(_NB // _TCB,),
        in_specs=[pl.BlockSpec((_TCB, _NR, _NCOL), lambda i: (i, 0, 0))],
        out_specs=pl.BlockSpec((_TCB, _NR, _NCOL), lambda i: (i, 0, 0)),
        out_shape=jax.ShapeDtypeStruct(x.shape, x.dtype),
    )(x)

    x5 = x.reshape(_NTOUCH, _RS, _NCOL)
    out_r = jax.new_ref(out0.reshape(_NTOUCH, _RS, _NCOL))

    run = pl.kernel(
        _sc_body,
        out_type=(),
        mesh=plsc.VectorSubcoreMesh(core_axis_name="c", subcore_axis_name="s"),
        compiler_params=pltpu.CompilerParams(needs_layout_passes=False),
        scratch_types=(
            [pltpu.VMEM((_NBUF, _CH, _NCOL), jnp.float32)]
            + [pltpu.SemaphoreType.DMA] * (2 * _NBUF)
        ),
    )
    run(x5, out_r)
    return out_r[...].reshape(x.shape)
